# baseline (speedup 1.0000x reference)
import functools

import jax
import jax.numpy as jnp
from jax import lax
from jax.experimental import pallas as pl
from jax.experimental.pallas import tpu as pltpu
from jax.experimental.pallas import tpu_sc as plsc

_G = 10000
_NC = 2
_NS = 16
_NW = _NC * _NS


def _mlp1_body(x_ref, w1_ref, b1_ref, o_ref):
    d = o_ref.shape[-1]
    h = jnp.dot(x_ref[...], w1_ref[...], preferred_element_type=jnp.float32)
    h = h + b1_ref[...]
    o_ref[...] = jax.nn.sigmoid(h[:, :d]) * h[:, d:]


def _mlp1(x, W1, b1):
    n, d = x.shape
    bn = 2000
    return pl.pallas_call(
        _mlp1_body,
        grid=(n // bn,),
        in_specs=[
            pl.BlockSpec((bn, d), lambda i: (i, 0)),
            pl.BlockSpec((d, 2 * d), lambda i: (0, 0)),
            pl.BlockSpec((1, 2 * d), lambda i: (0, 0)),
        ],
        out_specs=pl.BlockSpec((bn, d), lambda i: (i, 0)),
        out_shape=jax.ShapeDtypeStruct((n, d), jnp.float32),
    )(x, W1, b1.reshape(1, -1))


def _segment_sum_sc(vals, ids, init, gpad):
    n, d = vals.shape
    rows_per_w = n // _NW
    chunk = 40
    n_chunks = rows_per_w // chunk
    nbuf = 5
    n_groups = n_chunks // nbuf
    zrows = gpad // _NS
    mesh = plsc.VectorSubcoreMesh(core_axis_name="c", subcore_axis_name="s")

    @functools.partial(
        pl.kernel,
        out_type=jax.ShapeDtypeStruct((_NC * gpad, d), jnp.float32),
        mesh=mesh,
        scratch_types=[
            pltpu.VMEM((nbuf, chunk), jnp.int32),
            pltpu.VMEM((nbuf, chunk, d), jnp.float32),
            pltpu.VMEM_SHARED((gpad, d), jnp.float32),
            pltpu.SemaphoreType.DMA((nbuf,)),
            pltpu.SemaphoreType.DMA((nbuf,)),
            pltpu.SemaphoreType.DMA((nbuf,)),
        ],
    )
    def seg_kernel(vals_hbm, ids_hbm, z_hbm, out_hbm,
                   idx_v, rows_v, acc_sh, sem_i, sem_l, sem_s):
        cid = lax.axis_index("c")
        sid = lax.axis_index("s")
        wid = cid * _NS + sid
        pltpu.sync_copy(z_hbm.at[pl.ds(cid * gpad + sid * zrows, zrows)],
                        acc_sh.at[pl.ds(sid * zrows, zrows)])
        plsc.subcore_barrier()

        base = wid * rows_per_w

        def load(i, b, start):
            iargs = (ids_hbm.at[pl.ds(base + i * chunk, chunk)],
                     idx_v.at[b], sem_i.at[b])
            rargs = (vals_hbm.at[pl.ds(base + i * chunk, chunk)],
                     rows_v.at[b], sem_l.at[b])
            if start:
                pltpu.async_copy(*iargs)
                pltpu.async_copy(*rargs)
            else:
                pltpu.make_async_copy(*iargs).wait()
                pltpu.make_async_copy(*rargs).wait()

        def scat(i, b, start):
            args = (rows_v.at[b], acc_sh.at[idx_v.at[b]], sem_s.at[b])
            if start:
                return pltpu.async_copy(*args, add=True)
            return pltpu.make_async_copy(*args)

        for b in range(3):
            load(b, b, True)

        def group(j, carry):
            for b in range(nbuf):
                i = j * nbuf + b
                load(i, b, False)
                scat(i, b, True)
                k = i + 3
                kb = (b + 3) % nbuf

                @pl.when(jnp.logical_and(k < n_chunks, k >= nbuf))
                def _wait_prev_scatter():
                    scat(k - nbuf, kb, False).wait()

                @pl.when(k < n_chunks)
                def _load_ahead():
                    load(k, kb, True)
            return carry

        lax.fori_loop(0, n_groups, group, 0)
        for b in range(nbuf):
            scat((n_groups - 1) * nbuf + b, b, False).wait()
        plsc.subcore_barrier()

        out_off = cid * gpad + sid * zrows
        pltpu.sync_copy(acc_sh.at[pl.ds(sid * zrows, zrows)],
                        out_hbm.at[pl.ds(out_off, zrows)])

    return seg_kernel(vals, ids, init)


def _mlp2_body(p0_ref, p1_ref, w2_ref, b2_ref, o_ref):
    p = p0_ref[...] + p1_ref[...]
    o_ref[...] = (
        jnp.dot(p, w2_ref[...], preferred_element_type=jnp.float32)
        + b2_ref[...]
    )


def _mlp2(p0, p1, W2, b2):
    g, d = p0.shape
    bg = 2000
    return pl.pallas_call(
        _mlp2_body,
        grid=(g // bg,),
        in_specs=[
            pl.BlockSpec((bg, d), lambda i: (i, 0)),
            pl.BlockSpec((bg, d), lambda i: (i, 0)),
            pl.BlockSpec((d, d), lambda i: (0, 0)),
            pl.BlockSpec((1, d), lambda i: (0, 0)),
        ],
        out_specs=pl.BlockSpec((bg, d), lambda i: (i, 0)),
        out_shape=jax.ShapeDtypeStruct((g, d), jnp.float32),
    )(p0, p1, W2, b2.reshape(1, -1))


def kernel(node_states, graph_idx, n_graphs, W1, b1, W2, b2):
    n, d = node_states.shape
    ids = jnp.minimum(graph_idx, n_graphs - 1).astype(jnp.int32)
    gpad = _NS * 632
    nslab = 2
    sn = n // nslab
    p = jnp.zeros((_NC * gpad, d), jnp.float32)
    for s in range(nslab):
        vals = _mlp1(node_states[s * sn:(s + 1) * sn], W1, b1)
        p = _segment_sum_sc(vals, ids[s * sn:(s + 1) * sn], p, gpad)
    return _mlp2(p[:_G], p[gpad:gpad + _G], W2, b2)


# speedup vs baseline: 1.3531x; 1.3531x over previous
import functools

import jax
import jax.numpy as jnp
from jax import lax
from jax.experimental import pallas as pl
from jax.experimental.pallas import tpu as pltpu
from jax.experimental.pallas import tpu_sc as plsc

_G = 10000
_NC = 2
_NS = 16
_NW = _NC * _NS


def _mlp1_body(x_ref, w1_ref, b1_ref, o_ref):
    d = o_ref.shape[-1]
    h = jnp.dot(x_ref[...], w1_ref[...], preferred_element_type=jnp.float32)
    h = h + b1_ref[...]
    o_ref[...] = jax.nn.sigmoid(h[:, :d]) * h[:, d:]


def _mlp1(x, W1, b1, slab, nslab):
    n, d = x.shape
    sn = n // nslab
    bn = 2000
    boff = slab * (sn // bn)
    return pl.pallas_call(
        _mlp1_body,
        grid=(sn // bn,),
        in_specs=[
            pl.BlockSpec((bn, d), lambda i: (i + boff, 0)),
            pl.BlockSpec((d, 2 * d), lambda i: (0, 0)),
            pl.BlockSpec((1, 2 * d), lambda i: (0, 0)),
        ],
        out_specs=pl.BlockSpec((bn, d), lambda i: (i, 0)),
        out_shape=jax.ShapeDtypeStruct((sn, d), jnp.float32),
    )(x, W1, b1.reshape(1, -1))


def _segment_sum_sc(vals, ids, init, gpad, id_base):
    n, d = vals.shape
    rows_per_w = n // _NW
    chunk = 40
    n_chunks = rows_per_w // chunk
    nbuf = 5
    n_groups = n_chunks // nbuf
    zrows = gpad // _NS
    mesh = plsc.VectorSubcoreMesh(core_axis_name="c", subcore_axis_name="s")

    @functools.partial(
        pl.kernel,
        out_type=jax.ShapeDtypeStruct((_NC * gpad, d), jnp.float32),
        mesh=mesh,
        scratch_types=[
            pltpu.VMEM((nbuf, chunk), jnp.int32),
            pltpu.VMEM((nbuf, chunk, d), jnp.float32),
            pltpu.VMEM_SHARED((gpad, d), jnp.float32),
            pltpu.SemaphoreType.DMA((nbuf,)),
            pltpu.SemaphoreType.DMA((nbuf,)),
            pltpu.SemaphoreType.DMA((nbuf,)),
        ],
    )
    def seg_kernel(vals_hbm, ids_hbm, z_hbm, out_hbm,
                   idx_v, rows_v, acc_sh, sem_i, sem_l, sem_s):
        cid = lax.axis_index("c")
        sid = lax.axis_index("s")
        wid = cid * _NS + sid
        pltpu.sync_copy(z_hbm.at[pl.ds(cid * gpad + sid * zrows, zrows)],
                        acc_sh.at[pl.ds(sid * zrows, zrows)])
        plsc.subcore_barrier()

        base = wid * rows_per_w

        def load(i, b, start):
            iargs = (ids_hbm.at[pl.ds(id_base + base + i * chunk, chunk)],
                     idx_v.at[b], sem_i.at[b])
            rargs = (vals_hbm.at[pl.ds(base + i * chunk, chunk)],
                     rows_v.at[b], sem_l.at[b])
            if start:
                pltpu.async_copy(*iargs)
                pltpu.async_copy(*rargs)
            else:
                pltpu.make_async_copy(*iargs).wait()
                pltpu.make_async_copy(*rargs).wait()

        def scat(i, b, start):
            args = (rows_v.at[b], acc_sh.at[idx_v.at[b]], sem_s.at[b])
            if start:
                return pltpu.async_copy(*args, add=True)
            return pltpu.make_async_copy(*args)

        for b in range(3):
            load(b, b, True)

        def group(j, carry):
            for b in range(nbuf):
                i = j * nbuf + b
                load(i, b, False)
                scat(i, b, True)
                k = i + 3
                kb = (b + 3) % nbuf

                @pl.when(jnp.logical_and(k < n_chunks, k >= nbuf))
                def _wait_prev_scatter():
                    scat(k - nbuf, kb, False).wait()

                @pl.when(k < n_chunks)
                def _load_ahead():
                    load(k, kb, True)
            return carry

        lax.fori_loop(0, n_groups, group, 0)
        for b in range(nbuf):
            scat((n_groups - 1) * nbuf + b, b, False).wait()
        plsc.subcore_barrier()

        out_off = cid * gpad + sid * zrows
        pltpu.sync_copy(acc_sh.at[pl.ds(sid * zrows, zrows)],
                        out_hbm.at[pl.ds(out_off, zrows)])

    return seg_kernel(vals, ids, init)


def _mlp2_body(p0_ref, p1_ref, w2_ref, b2_ref, o_ref):
    p = p0_ref[...] + p1_ref[...]
    o_ref[...] = (
        jnp.dot(p, w2_ref[...], preferred_element_type=jnp.float32)
        + b2_ref[...]
    )


def _mlp2(p0, p1, W2, b2):
    g, d = p0.shape
    bg = 2000
    return pl.pallas_call(
        _mlp2_body,
        grid=(g // bg,),
        in_specs=[
            pl.BlockSpec((bg, d), lambda i: (i, 0)),
            pl.BlockSpec((bg, d), lambda i: (i, 0)),
            pl.BlockSpec((d, d), lambda i: (0, 0)),
            pl.BlockSpec((1, d), lambda i: (0, 0)),
        ],
        out_specs=pl.BlockSpec((bg, d), lambda i: (i, 0)),
        out_shape=jax.ShapeDtypeStruct((g, d), jnp.float32),
    )(p0, p1, W2, b2.reshape(1, -1))


def kernel(node_states, graph_idx, n_graphs, W1, b1, W2, b2):
    n, d = node_states.shape
    ids = jnp.minimum(graph_idx, n_graphs - 1).astype(jnp.int32)
    gpad = _NS * 632
    nslab = 2
    sn = n // nslab
    p = jnp.zeros((_NC * gpad, d), jnp.float32)
    for s in range(nslab):
        vals = _mlp1(node_states, W1, b1, s, nslab)
        p = _segment_sum_sc(vals, ids, p, gpad, s * sn)
    return _mlp2(p[:_G], p[gpad:gpad + _G], W2, b2)


# speedup vs baseline: 1.3536x; 1.0003x over previous
import functools

import jax
import jax.numpy as jnp
from jax import lax
from jax.experimental import pallas as pl
from jax.experimental.pallas import tpu as pltpu
from jax.experimental.pallas import tpu_sc as plsc

_G = 10000
_NC = 2
_NS = 16
_NW = _NC * _NS


def _mlp1_body(x_ref, w1_ref, b1_ref, o_ref):
    d = o_ref.shape[-1]
    h = jnp.dot(x_ref[...], w1_ref[...], preferred_element_type=jnp.float32)
    h = h + b1_ref[...]
    o_ref[...] = jax.nn.sigmoid(h[:, :d]) * h[:, d:]


def _mlp1(x, W1, b1, slab, nslab):
    n, d = x.shape
    sn = n // nslab
    bn = 2000
    boff = slab * (sn // bn)
    return pl.pallas_call(
        _mlp1_body,
        grid=(sn // bn,),
        in_specs=[
            pl.BlockSpec((bn, d), lambda i: (i + boff, 0)),
            pl.BlockSpec((d, 2 * d), lambda i: (0, 0)),
            pl.BlockSpec((1, 2 * d), lambda i: (0, 0)),
        ],
        out_specs=pl.BlockSpec((bn, d), lambda i: (i, 0)),
        out_shape=jax.ShapeDtypeStruct((sn, d), jnp.float32),
    )(x, W1, b1.reshape(1, -1))


def _segment_sum_sc(vals, ids, init, gpad, id_base):
    n, d = vals.shape
    rows_per_w = n // _NW
    chunk = 40
    n_chunks = rows_per_w // chunk
    nbuf = 5
    n_groups = n_chunks // nbuf
    zrows = gpad // _NS
    mesh = plsc.VectorSubcoreMesh(core_axis_name="c", subcore_axis_name="s")

    @functools.partial(
        pl.kernel,
        out_type=jax.ShapeDtypeStruct((_NC * gpad, d), jnp.float32),
        mesh=mesh,
        scratch_types=[
            pltpu.VMEM((nbuf, chunk), jnp.int32),
            pltpu.VMEM((nbuf, chunk, d), jnp.float32),
            pltpu.VMEM_SHARED((gpad, d), jnp.float32),
            pltpu.SemaphoreType.DMA((nbuf,)),
            pltpu.SemaphoreType.DMA((nbuf,)),
            pltpu.SemaphoreType.DMA((nbuf,)),
        ],
    )
    def seg_kernel(vals_hbm, ids_hbm, z_hbm, out_hbm,
                   idx_v, rows_v, acc_sh, sem_i, sem_l, sem_s):
        cid = lax.axis_index("c")
        sid = lax.axis_index("s")
        wid = cid * _NS + sid
        pltpu.sync_copy(z_hbm.at[pl.ds(cid * gpad + sid * zrows, zrows)],
                        acc_sh.at[pl.ds(sid * zrows, zrows)])
        plsc.subcore_barrier()

        base = wid * rows_per_w

        def load(i, b, start):
            iargs = (ids_hbm.at[pl.ds(id_base + base + i * chunk, chunk)],
                     idx_v.at[b], sem_i.at[b])
            rargs = (vals_hbm.at[pl.ds(base + i * chunk, chunk)],
                     rows_v.at[b], sem_l.at[b])
            if start:
                pltpu.async_copy(*iargs)
                pltpu.async_copy(*rargs)
            else:
                pltpu.make_async_copy(*iargs).wait()
                pltpu.make_async_copy(*rargs).wait()

        def scat(i, b, start):
            args = (rows_v.at[b], acc_sh.at[idx_v.at[b]], sem_s.at[b])
            if start:
                return pltpu.async_copy(*args, add=True)
            return pltpu.make_async_copy(*args)

        for b in range(3):
            load(b, b, True)

        def group(j, carry):
            for b in range(nbuf):
                i = j * nbuf + b
                load(i, b, False)
                scat(i, b, True)
                k = i + 3
                kb = (b + 3) % nbuf

                @pl.when(jnp.logical_and(k < n_chunks, k >= nbuf))
                def _wait_prev_scatter():
                    scat(k - nbuf, kb, False).wait()

                @pl.when(k < n_chunks)
                def _load_ahead():
                    load(k, kb, True)
            return carry

        lax.fori_loop(0, n_groups, group, 0)
        for b in range(nbuf):
            scat((n_groups - 1) * nbuf + b, b, False).wait()
        plsc.subcore_barrier()

        out_off = cid * gpad + sid * zrows
        pltpu.sync_copy(acc_sh.at[pl.ds(sid * zrows, zrows)],
                        out_hbm.at[pl.ds(out_off, zrows)])

    return seg_kernel(vals, ids, init)


def _mlp2_body(p0_ref, p1_ref, w2_ref, b2_ref, o_ref):
    p = p0_ref[...] + p1_ref[...]
    o_ref[...] = (
        jnp.dot(p, w2_ref[...], preferred_element_type=jnp.float32)
        + b2_ref[...]
    )


def _mlp2(p0, p1, W2, b2):
    g, d = p0.shape
    bg = 2000
    return pl.pallas_call(
        _mlp2_body,
        grid=(g // bg,),
        in_specs=[
            pl.BlockSpec((bg, d), lambda i: (i, 0)),
            pl.BlockSpec((bg, d), lambda i: (i, 0)),
            pl.BlockSpec((d, d), lambda i: (0, 0)),
            pl.BlockSpec((1, d), lambda i: (0, 0)),
        ],
        out_specs=pl.BlockSpec((bg, d), lambda i: (i, 0)),
        out_shape=jax.ShapeDtypeStruct((g, d), jnp.float32),
    )(p0, p1, W2, b2.reshape(1, -1))


def kernel(node_states, graph_idx, n_graphs, W1, b1, W2, b2):
    n, d = node_states.shape
    ids = jnp.minimum(graph_idx, n_graphs - 1).astype(jnp.int32)
    gpad = _NS * 632
    nslab = 2
    sn = n // nslab
    p = jnp.zeros((_NC * gpad, d), jnp.float32)
    vals = [_mlp1(node_states, W1, b1, s, nslab) for s in range(nslab)]
    for s in range(nslab):
        p = _segment_sum_sc(vals[s], ids, p, gpad, s * sn)
    return _mlp2(p[:_G], p[gpad:gpad + _G], W2, b2)


# speedup vs baseline: 1.5223x; 1.1247x over previous
import functools

import jax
import jax.numpy as jnp
from jax import lax
from jax.experimental import pallas as pl
from jax.experimental.pallas import tpu as pltpu
from jax.experimental.pallas import tpu_sc as plsc

_G = 10000
_NC = 2
_NS = 16
_NW = _NC * _NS


def _mlp1_body(x_ref, w1_ref, b1_ref, o_ref):
    d = o_ref.shape[-1]
    h = jnp.dot(x_ref[...], w1_ref[...], preferred_element_type=jnp.float32)
    h = h + b1_ref[...]
    o_ref[...] = jax.nn.sigmoid(h[:, :d]) * h[:, d:]


def _mlp1(x, W1, b1, slab, nslab):
    n, d = x.shape
    sn = n // nslab
    bn = 5000
    boff = slab * (sn // bn)
    return pl.pallas_call(
        _mlp1_body,
        grid=(sn // bn,),
        in_specs=[
            pl.BlockSpec((bn, d), lambda i: (i + boff, 0)),
            pl.BlockSpec((d, 2 * d), lambda i: (0, 0)),
            pl.BlockSpec((1, 2 * d), lambda i: (0, 0)),
        ],
        out_specs=pl.BlockSpec((bn, d), lambda i: (i, 0)),
        out_shape=jax.ShapeDtypeStruct((sn, d), jnp.float32),
    )(x, W1, b1.reshape(1, -1))


def _segment_sum_sc(vals, ids, init, gpad, id_base):
    n, d = vals.shape
    rows_per_w = n // _NW
    chunk = 40
    n_chunks = rows_per_w // chunk
    nbuf = 5
    n_groups = n_chunks // nbuf
    zrows = gpad // _NS
    mesh = plsc.VectorSubcoreMesh(core_axis_name="c", subcore_axis_name="s")

    @functools.partial(
        pl.kernel,
        out_type=jax.ShapeDtypeStruct((_NC * gpad, d), jnp.float32),
        mesh=mesh,
        scratch_types=[
            pltpu.VMEM((nbuf, chunk), jnp.int32),
            pltpu.VMEM((nbuf, chunk, d), jnp.float32),
            pltpu.VMEM_SHARED((gpad, d), jnp.float32),
            pltpu.SemaphoreType.DMA((nbuf,)),
            pltpu.SemaphoreType.DMA((nbuf,)),
            pltpu.SemaphoreType.DMA((nbuf,)),
        ],
    )
    def seg_kernel(vals_hbm, ids_hbm, z_hbm, out_hbm,
                   idx_v, rows_v, acc_sh, sem_i, sem_l, sem_s):
        cid = lax.axis_index("c")
        sid = lax.axis_index("s")
        wid = cid * _NS + sid
        pltpu.sync_copy(z_hbm.at[pl.ds(cid * gpad + sid * zrows, zrows)],
                        acc_sh.at[pl.ds(sid * zrows, zrows)])
        plsc.subcore_barrier()

        base = wid * rows_per_w

        def load(i, b, start):
            iargs = (ids_hbm.at[pl.ds(id_base + base + i * chunk, chunk)],
                     idx_v.at[b], sem_i.at[b])
            rargs = (vals_hbm.at[pl.ds(base + i * chunk, chunk)],
                     rows_v.at[b], sem_l.at[b])
            if start:
                pltpu.async_copy(*iargs)
                pltpu.async_copy(*rargs)
            else:
                pltpu.make_async_copy(*iargs).wait()
                pltpu.make_async_copy(*rargs).wait()

        def scat(i, b, start):
            args = (rows_v.at[b], acc_sh.at[idx_v.at[b]], sem_s.at[b])
            if start:
                return pltpu.async_copy(*args, add=True)
            return pltpu.make_async_copy(*args)

        for b in range(3):
            load(b, b, True)

        def group(j, carry):
            for b in range(nbuf):
                i = j * nbuf + b
                load(i, b, False)
                scat(i, b, True)
                k = i + 3
                kb = (b + 3) % nbuf

                @pl.when(jnp.logical_and(k < n_chunks, k >= nbuf))
                def _wait_prev_scatter():
                    scat(k - nbuf, kb, False).wait()

                @pl.when(k < n_chunks)
                def _load_ahead():
                    load(k, kb, True)
            return carry

        lax.fori_loop(0, n_groups, group, 0)
        for b in range(nbuf):
            scat((n_groups - 1) * nbuf + b, b, False).wait()
        plsc.subcore_barrier()

        out_off = cid * gpad + sid * zrows
        pltpu.sync_copy(acc_sh.at[pl.ds(sid * zrows, zrows)],
                        out_hbm.at[pl.ds(out_off, zrows)])

    return seg_kernel(vals, ids, init)


def _mlp2_body(p0_ref, p1_ref, w2_ref, b2_ref, o_ref):
    p = p0_ref[...] + p1_ref[...]
    o_ref[...] = (
        jnp.dot(p, w2_ref[...], preferred_element_type=jnp.float32)
        + b2_ref[...]
    )


def _mlp2(p0, p1, W2, b2):
    g, d = p0.shape
    bg = 2000
    return pl.pallas_call(
        _mlp2_body,
        grid=(g // bg,),
        in_specs=[
            pl.BlockSpec((bg, d), lambda i: (i, 0)),
            pl.BlockSpec((bg, d), lambda i: (i, 0)),
            pl.BlockSpec((d, d), lambda i: (0, 0)),
            pl.BlockSpec((1, d), lambda i: (0, 0)),
        ],
        out_specs=pl.BlockSpec((bg, d), lambda i: (i, 0)),
        out_shape=jax.ShapeDtypeStruct((g, d), jnp.float32),
    )(p0, p1, W2, b2.reshape(1, -1))


def kernel(node_states, graph_idx, n_graphs, W1, b1, W2, b2):
    n, d = node_states.shape
    ids = jnp.minimum(graph_idx, n_graphs - 1).astype(jnp.int32)
    gpad = _NS * 632
    nslab = 2
    sn = n // nslab
    p = jnp.zeros((_NC * gpad, d), jnp.float32)
    vals = [_mlp1(node_states, W1, b1, s, nslab) for s in range(nslab)]
    for s in range(nslab):
        p = _segment_sum_sc(vals[s], ids, p, gpad, s * sn)
    return _mlp2(p[:_G], p[gpad:gpad + _G], W2, b2)


# speedup vs baseline: 1.5608x; 1.0253x over previous
import functools

import jax
import jax.numpy as jnp
from jax import lax
from jax.experimental import pallas as pl
from jax.experimental.pallas import tpu as pltpu
from jax.experimental.pallas import tpu_sc as plsc

_G = 10000
_NC = 2
_NS = 16
_NW = _NC * _NS


def _mlp1_body(x_ref, w1_ref, b1_ref, o_ref):
    d = o_ref.shape[-1]
    h = jnp.dot(x_ref[...], w1_ref[...], preferred_element_type=jnp.float32)
    h = h + b1_ref[...]
    o_ref[...] = jax.nn.sigmoid(h[:, :d]) * h[:, d:]


def _mlp1(x, W1, b1, slab, nslab):
    n, d = x.shape
    sn = n // nslab
    bn = 10000
    boff = slab * (sn // bn)
    return pl.pallas_call(
        _mlp1_body,
        grid=(sn // bn,),
        in_specs=[
            pl.BlockSpec((bn, d), lambda i: (i + boff, 0)),
            pl.BlockSpec((d, 2 * d), lambda i: (0, 0)),
            pl.BlockSpec((1, 2 * d), lambda i: (0, 0)),
        ],
        out_specs=pl.BlockSpec((bn, d), lambda i: (i, 0)),
        out_shape=jax.ShapeDtypeStruct((sn, d), jnp.float32),
    )(x, W1, b1.reshape(1, -1))


def _segment_sum_sc(vals, ids, init, gpad, id_base):
    n, d = vals.shape
    rows_per_w = n // _NW
    chunk = 40
    n_chunks = rows_per_w // chunk
    nbuf = 5
    n_groups = n_chunks // nbuf
    zrows = gpad // _NS
    mesh = plsc.VectorSubcoreMesh(core_axis_name="c", subcore_axis_name="s")

    @functools.partial(
        pl.kernel,
        out_type=jax.ShapeDtypeStruct((_NC * gpad, d), jnp.float32),
        mesh=mesh,
        scratch_types=[
            pltpu.VMEM((nbuf, chunk), jnp.int32),
            pltpu.VMEM((nbuf, chunk, d), jnp.float32),
            pltpu.VMEM_SHARED((gpad, d), jnp.float32),
            pltpu.SemaphoreType.DMA((nbuf,)),
            pltpu.SemaphoreType.DMA((nbuf,)),
            pltpu.SemaphoreType.DMA((nbuf,)),
        ],
    )
    def seg_kernel(vals_hbm, ids_hbm, z_hbm, out_hbm,
                   idx_v, rows_v, acc_sh, sem_i, sem_l, sem_s):
        cid = lax.axis_index("c")
        sid = lax.axis_index("s")
        wid = cid * _NS + sid
        pltpu.sync_copy(z_hbm.at[pl.ds(cid * gpad + sid * zrows, zrows)],
                        acc_sh.at[pl.ds(sid * zrows, zrows)])
        plsc.subcore_barrier()

        base = wid * rows_per_w

        def load(i, b, start):
            iargs = (ids_hbm.at[pl.ds(id_base + base + i * chunk, chunk)],
                     idx_v.at[b], sem_i.at[b])
            rargs = (vals_hbm.at[pl.ds(base + i * chunk, chunk)],
                     rows_v.at[b], sem_l.at[b])
            if start:
                pltpu.async_copy(*iargs)
                pltpu.async_copy(*rargs)
            else:
                pltpu.make_async_copy(*iargs).wait()
                pltpu.make_async_copy(*rargs).wait()

        def scat(i, b, start):
            args = (rows_v.at[b], acc_sh.at[idx_v.at[b]], sem_s.at[b])
            if start:
                return pltpu.async_copy(*args, add=True)
            return pltpu.make_async_copy(*args)

        for b in range(3):
            load(b, b, True)

        def group(j, carry):
            for b in range(nbuf):
                i = j * nbuf + b
                load(i, b, False)
                scat(i, b, True)
                k = i + 3
                kb = (b + 3) % nbuf

                @pl.when(jnp.logical_and(k < n_chunks, k >= nbuf))
                def _wait_prev_scatter():
                    scat(k - nbuf, kb, False).wait()

                @pl.when(k < n_chunks)
                def _load_ahead():
                    load(k, kb, True)
            return carry

        lax.fori_loop(0, n_groups, group, 0)
        for b in range(nbuf):
            scat((n_groups - 1) * nbuf + b, b, False).wait()
        plsc.subcore_barrier()

        out_off = cid * gpad + sid * zrows
        pltpu.sync_copy(acc_sh.at[pl.ds(sid * zrows, zrows)],
                        out_hbm.at[pl.ds(out_off, zrows)])

    return seg_kernel(vals, ids, init)


def _mlp2_body(p0_ref, p1_ref, w2_ref, b2_ref, o_ref):
    p = p0_ref[...] + p1_ref[...]
    o_ref[...] = (
        jnp.dot(p, w2_ref[...], preferred_element_type=jnp.float32)
        + b2_ref[...]
    )


def _mlp2(p0, p1, W2, b2):
    g, d = p0.shape
    bg = 2000
    return pl.pallas_call(
        _mlp2_body,
        grid=(g // bg,),
        in_specs=[
            pl.BlockSpec((bg, d), lambda i: (i, 0)),
            pl.BlockSpec((bg, d), lambda i: (i, 0)),
            pl.BlockSpec((d, d), lambda i: (0, 0)),
            pl.BlockSpec((1, d), lambda i: (0, 0)),
        ],
        out_specs=pl.BlockSpec((bg, d), lambda i: (i, 0)),
        out_shape=jax.ShapeDtypeStruct((g, d), jnp.float32),
    )(p0, p1, W2, b2.reshape(1, -1))


def kernel(node_states, graph_idx, n_graphs, W1, b1, W2, b2):
    n, d = node_states.shape
    ids = jnp.minimum(graph_idx, n_graphs - 1).astype(jnp.int32)
    gpad = _NS * 632
    nslab = 2
    sn = n // nslab
    p = jnp.zeros((_NC * gpad, d), jnp.float32)
    vals = [_mlp1(node_states, W1, b1, s, nslab) for s in range(nslab)]
    for s in range(nslab):
        p = _segment_sum_sc(vals[s], ids, p, gpad, s * sn)
    return _mlp2(p[:_G], p[gpad:gpad + _G], W2, b2)


# speedup vs baseline: 1.5754x; 1.0094x over previous
import functools

import jax
import jax.numpy as jnp
from jax import lax
from jax.experimental import pallas as pl
from jax.experimental.pallas import tpu as pltpu
from jax.experimental.pallas import tpu_sc as plsc

_G = 10000
_NC = 2
_NS = 16
_NW = _NC * _NS


def _mlp1_body(x_ref, w1_ref, b1_ref, o_ref):
    d = o_ref.shape[-1]
    h = jnp.dot(x_ref[...], w1_ref[...], preferred_element_type=jnp.float32)
    h = h + b1_ref[...]
    o_ref[...] = jax.nn.sigmoid(h[:, :d]) * h[:, d:]


def _mlp1(x, W1, b1, slab, nslab):
    n, d = x.shape
    sn = n // nslab
    bn = 10000
    boff = slab * (sn // bn)
    return pl.pallas_call(
        _mlp1_body,
        grid=(sn // bn,),
        in_specs=[
            pl.BlockSpec((bn, d), lambda i: (i + boff, 0)),
            pl.BlockSpec((d, 2 * d), lambda i: (0, 0)),
            pl.BlockSpec((1, 2 * d), lambda i: (0, 0)),
        ],
        out_specs=pl.BlockSpec((bn, d), lambda i: (i, 0)),
        out_shape=jax.ShapeDtypeStruct((sn, d), jnp.float32),
    )(x, W1, b1.reshape(1, -1))


def _segment_sum_sc(vals, ids, init, gpad, id_base):
    n, d = vals.shape
    rows_per_w = n // _NW
    chunk = 40
    n_chunks = rows_per_w // chunk
    nbuf = 5
    n_groups = n_chunks // nbuf
    zrows = gpad // _NS
    mesh = plsc.VectorSubcoreMesh(core_axis_name="c", subcore_axis_name="s")

    @functools.partial(
        pl.kernel,
        out_type=jax.ShapeDtypeStruct((_NC * gpad, d), jnp.float32),
        mesh=mesh,
        scratch_types=[
            pltpu.VMEM((nbuf, chunk), jnp.int32),
            pltpu.VMEM((nbuf, chunk, d), jnp.float32),
            pltpu.VMEM_SHARED((gpad, d), jnp.float32),
            pltpu.SemaphoreType.DMA((nbuf,)),
            pltpu.SemaphoreType.DMA((nbuf,)),
            pltpu.SemaphoreType.DMA((nbuf,)),
        ],
    )
    def seg_kernel(vals_hbm, ids_hbm, z_hbm, out_hbm,
                   idx_v, rows_v, acc_sh, sem_i, sem_l, sem_s):
        cid = lax.axis_index("c")
        sid = lax.axis_index("s")
        wid = cid * _NS + sid
        pltpu.sync_copy(z_hbm.at[pl.ds(cid * gpad + sid * zrows, zrows)],
                        acc_sh.at[pl.ds(sid * zrows, zrows)])
        plsc.subcore_barrier()

        base = wid * rows_per_w

        def load(i, b, start):
            iargs = (ids_hbm.at[pl.ds(id_base + base + i * chunk, chunk)],
                     idx_v.at[b], sem_i.at[b])
            rargs = (vals_hbm.at[pl.ds(base + i * chunk, chunk)],
                     rows_v.at[b], sem_l.at[b])
            if start:
                pltpu.async_copy(*iargs)
                pltpu.async_copy(*rargs)
            else:
                pltpu.make_async_copy(*iargs).wait()
                pltpu.make_async_copy(*rargs).wait()

        def scat(i, b, start):
            args = (rows_v.at[b], acc_sh.at[idx_v.at[b]], sem_s.at[b])
            if start:
                return pltpu.async_copy(*args, add=True)
            return pltpu.make_async_copy(*args)

        for b in range(3):
            load(b, b, True)

        def group(j, carry):
            for b in range(nbuf):
                i = j * nbuf + b
                load(i, b, False)
                scat(i, b, True)
                k = i + 3
                kb = (b + 3) % nbuf

                @pl.when(jnp.logical_and(k < n_chunks, k >= nbuf))
                def _wait_prev_scatter():
                    scat(k - nbuf, kb, False).wait()

                @pl.when(k < n_chunks)
                def _load_ahead():
                    load(k, kb, True)
            return carry

        lax.fori_loop(0, n_groups, group, 0)
        for b in range(nbuf):
            scat((n_groups - 1) * nbuf + b, b, False).wait()
        plsc.subcore_barrier()

        out_off = cid * gpad + sid * zrows
        pltpu.sync_copy(acc_sh.at[pl.ds(sid * zrows, zrows)],
                        out_hbm.at[pl.ds(out_off, zrows)])

    return seg_kernel(vals, ids, init)


def _mlp2_body(p0_ref, p1_ref, w2_ref, b2_ref, o_ref):
    p = p0_ref[...] + p1_ref[...]
    o_ref[...] = (
        jnp.dot(p, w2_ref[...], preferred_element_type=jnp.float32)
        + b2_ref[...]
    )


def _mlp2(p0, p1, W2, b2):
    g, d = p0.shape
    bg = 2000
    return pl.pallas_call(
        _mlp2_body,
        grid=(g // bg,),
        in_specs=[
            pl.BlockSpec((bg, d), lambda i: (i, 0)),
            pl.BlockSpec((bg, d), lambda i: (i, 0)),
            pl.BlockSpec((d, d), lambda i: (0, 0)),
            pl.BlockSpec((1, d), lambda i: (0, 0)),
        ],
        out_specs=pl.BlockSpec((bg, d), lambda i: (i, 0)),
        out_shape=jax.ShapeDtypeStruct((g, d), jnp.float32),
    )(p0, p1, W2, b2.reshape(1, -1))


def kernel(node_states, graph_idx, n_graphs, W1, b1, W2, b2):
    n, d = node_states.shape
    ids = jnp.minimum(graph_idx, n_graphs - 1).astype(jnp.int32)
    gpad = _NS * 632
    nslab = 1
    sn = n // nslab
    p = jnp.zeros((_NC * gpad, d), jnp.float32)
    vals = [_mlp1(node_states, W1, b1, s, nslab) for s in range(nslab)]
    for s in range(nslab):
        p = _segment_sum_sc(vals[s], ids, p, gpad, s * sn)
    return _mlp2(p[:_G], p[gpad:gpad + _G], W2, b2)


# speedup vs baseline: 1.5925x; 1.0108x over previous
import functools

import jax
import jax.numpy as jnp
from jax import lax
from jax.experimental import pallas as pl
from jax.experimental.pallas import tpu as pltpu
from jax.experimental.pallas import tpu_sc as plsc

_G = 10000
_NC = 2
_NS = 16
_NW = _NC * _NS


def _mlp1_body(x_ref, w1_ref, b1_ref, o_ref):
    d = o_ref.shape[-1]
    h = jnp.dot(x_ref[...], w1_ref[...], preferred_element_type=jnp.float32)
    h = h + b1_ref[...]
    o_ref[...] = jax.nn.sigmoid(h[:, :d]) * h[:, d:]


def _mlp1(x, W1, b1, slab, nslab):
    n, d = x.shape
    sn = n // nslab
    bn = 16000
    boff = slab * (sn // bn)
    return pl.pallas_call(
        _mlp1_body,
        grid=(sn // bn,),
        in_specs=[
            pl.BlockSpec((bn, d), lambda i: (i + boff, 0)),
            pl.BlockSpec((d, 2 * d), lambda i: (0, 0)),
            pl.BlockSpec((1, 2 * d), lambda i: (0, 0)),
        ],
        out_specs=pl.BlockSpec((bn, d), lambda i: (i, 0)),
        out_shape=jax.ShapeDtypeStruct((sn, d), jnp.float32),
    )(x, W1, b1.reshape(1, -1))


def _segment_sum_sc(vals, ids, init, gpad, id_base):
    n, d = vals.shape
    rows_per_w = n // _NW
    chunk = 40
    n_chunks = rows_per_w // chunk
    nbuf = 5
    n_groups = n_chunks // nbuf
    zrows = gpad // _NS
    mesh = plsc.VectorSubcoreMesh(core_axis_name="c", subcore_axis_name="s")

    @functools.partial(
        pl.kernel,
        out_type=jax.ShapeDtypeStruct((_NC * gpad, d), jnp.float32),
        mesh=mesh,
        scratch_types=[
            pltpu.VMEM((nbuf, chunk), jnp.int32),
            pltpu.VMEM((nbuf, chunk, d), jnp.float32),
            pltpu.VMEM_SHARED((gpad, d), jnp.float32),
            pltpu.SemaphoreType.DMA((nbuf,)),
            pltpu.SemaphoreType.DMA((nbuf,)),
            pltpu.SemaphoreType.DMA((nbuf,)),
        ],
    )
    def seg_kernel(vals_hbm, ids_hbm, z_hbm, out_hbm,
                   idx_v, rows_v, acc_sh, sem_i, sem_l, sem_s):
        cid = lax.axis_index("c")
        sid = lax.axis_index("s")
        wid = cid * _NS + sid
        pltpu.sync_copy(z_hbm.at[pl.ds(cid * gpad + sid * zrows, zrows)],
                        acc_sh.at[pl.ds(sid * zrows, zrows)])
        plsc.subcore_barrier()

        base = wid * rows_per_w

        def load(i, b, start):
            iargs = (ids_hbm.at[pl.ds(id_base + base + i * chunk, chunk)],
                     idx_v.at[b], sem_i.at[b])
            rargs = (vals_hbm.at[pl.ds(base + i * chunk, chunk)],
                     rows_v.at[b], sem_l.at[b])
            if start:
                pltpu.async_copy(*iargs)
                pltpu.async_copy(*rargs)
            else:
                pltpu.make_async_copy(*iargs).wait()
                pltpu.make_async_copy(*rargs).wait()

        def scat(i, b, start):
            args = (rows_v.at[b], acc_sh.at[idx_v.at[b]], sem_s.at[b])
            if start:
                return pltpu.async_copy(*args, add=True)
            return pltpu.make_async_copy(*args)

        for b in range(3):
            load(b, b, True)

        def group(j, carry):
            for b in range(nbuf):
                i = j * nbuf + b
                load(i, b, False)
                scat(i, b, True)
                k = i + 3
                kb = (b + 3) % nbuf

                @pl.when(jnp.logical_and(k < n_chunks, k >= nbuf))
                def _wait_prev_scatter():
                    scat(k - nbuf, kb, False).wait()

                @pl.when(k < n_chunks)
                def _load_ahead():
                    load(k, kb, True)
            return carry

        lax.fori_loop(0, n_groups, group, 0)
        for b in range(nbuf):
            scat((n_groups - 1) * nbuf + b, b, False).wait()
        plsc.subcore_barrier()

        out_off = cid * gpad + sid * zrows
        pltpu.sync_copy(acc_sh.at[pl.ds(sid * zrows, zrows)],
                        out_hbm.at[pl.ds(out_off, zrows)])

    return seg_kernel(vals, ids, init)


def _mlp2_body(p0_ref, p1_ref, w2_ref, b2_ref, o_ref):
    p = p0_ref[...] + p1_ref[...]
    o_ref[...] = (
        jnp.dot(p, w2_ref[...], preferred_element_type=jnp.float32)
        + b2_ref[...]
    )


def _mlp2(p0, p1, W2, b2):
    g, d = p0.shape
    bg = 10000
    return pl.pallas_call(
        _mlp2_body,
        grid=(g // bg,),
        in_specs=[
            pl.BlockSpec((bg, d), lambda i: (i, 0)),
            pl.BlockSpec((bg, d), lambda i: (i, 0)),
            pl.BlockSpec((d, d), lambda i: (0, 0)),
            pl.BlockSpec((1, d), lambda i: (0, 0)),
        ],
        out_specs=pl.BlockSpec((bg, d), lambda i: (i, 0)),
        out_shape=jax.ShapeDtypeStruct((g, d), jnp.float32),
    )(p0, p1, W2, b2.reshape(1, -1))


def kernel(node_states, graph_idx, n_graphs, W1, b1, W2, b2):
    n, d = node_states.shape
    ids = jnp.minimum(graph_idx, n_graphs - 1).astype(jnp.int32)
    gpad = _NS * 632
    nslab = 1
    sn = n // nslab
    p = jnp.zeros((_NC * gpad, d), jnp.float32)
    vals = [_mlp1(node_states, W1, b1, s, nslab) for s in range(nslab)]
    for s in range(nslab):
        p = _segment_sum_sc(vals[s], ids, p, gpad, s * sn)
    return _mlp2(p[:_G], p[gpad:gpad + _G], W2, b2)
